# Initial kernel scaffold; baseline (speedup 1.0000x reference)
import jax
import jax.numpy as jnp
from jax.experimental import pallas as pl


def kernel(x, edge_index, W1, b1, W2, b2):
    raise NotImplementedError("write your pallas kernel here")


# speedup vs baseline: 51.8374x; 51.8374x over previous
import functools

import jax
import jax.numpy as jnp
from jax import lax
from jax.experimental import pallas as pl
from jax.experimental.pallas import tpu as pltpu
from jax.experimental.pallas import tpu_sc as plsc

N = 10000
D = 128
H = 16
E = 320000

NC = 2
NS = 16
NW = NC * NS
CHUNK = 128
CPW = 80
EPW = CPW * CHUNK
E_PAD = NW * EPW
NPAD = 10240
RPT = NPAD // NS
DUMMY = 10008

_mesh = plsc.VectorSubcoreMesh(core_axis_name="c", subcore_axis_name="s")


@functools.partial(
    pl.kernel,
    out_type=jax.ShapeDtypeStruct((NC, NPAD), jnp.float32),
    mesh=_mesh,
    scratch_types=[
        pltpu.VMEM_SHARED((NPAD,), jnp.float32),
        pltpu.VMEM((CPW, CHUNK), jnp.int32),
        pltpu.VMEM((CHUNK,), jnp.float32),
        pltpu.VMEM((RPT,), jnp.float32),
    ],
)
def _degree_kernel(dst_hbm, out_hbm, deg_sh, dst_v, ones_v, zero_v):
    c = lax.axis_index("c")
    s = lax.axis_index("s")
    w = s * NC + c
    one = jnp.ones((16,), jnp.float32)
    zero = jnp.zeros((16,), jnp.float32)
    for k in range(CHUNK // 16):
        ones_v[pl.ds(k * 16, 16)] = one
    for k in range(RPT // 16):
        zero_v[pl.ds(k * 16, 16)] = zero
    pltpu.sync_copy(dst_hbm.at[w], dst_v)
    pltpu.sync_copy(zero_v, deg_sh.at[pl.ds(s * RPT, RPT)])
    plsc.subcore_barrier()

    def body(j, carry):
        pltpu.sync_copy(ones_v, deg_sh.at[dst_v.at[j]], add=True)
        return carry

    lax.fori_loop(0, CPW, body, 0)
    plsc.subcore_barrier()
    pltpu.sync_copy(deg_sh.at[pl.ds(s * RPT, RPT)],
                    out_hbm.at[c, pl.ds(s * RPT, RPT)])


@functools.partial(
    pl.kernel,
    out_type=jax.ShapeDtypeStruct((NC, NPAD, H), jnp.float32),
    mesh=_mesh,
    scratch_types=[
        pltpu.VMEM_SHARED((NPAD, H), jnp.float32),
        pltpu.VMEM_SHARED((NPAD, H), jnp.float32),
        pltpu.VMEM((CPW, CHUNK), jnp.int32),
        pltpu.VMEM((CPW, CHUNK), jnp.int32),
        pltpu.VMEM((CHUNK, H), jnp.float32),
        pltpu.SemaphoreType.DMA,
    ],
    compiler_params=pltpu.CompilerParams(use_tc_tiling_on_sc=False),
)
def _edge_agg_kernel(q_hbm, src_hbm, dst_hbm, out_hbm,
                     agg_sh, q_sh, src_v, dst_v, rows_v, gsem):
    c = lax.axis_index("c")
    s = lax.axis_index("s")
    w = s * NC + c
    pltpu.sync_copy(q_hbm.at[pl.ds(s * RPT, RPT)],
                    agg_sh.at[pl.ds(s * RPT, RPT)])
    pltpu.sync_copy(q_hbm.at[pl.ds(s * RPT, RPT)],
                    q_sh.at[pl.ds(s * RPT, RPT)])
    pltpu.sync_copy(src_hbm.at[w], src_v)
    pltpu.sync_copy(dst_hbm.at[w], dst_v)
    plsc.subcore_barrier()

    def body(j, carry):
        pltpu.async_copy(q_sh.at[src_v.at[j]], rows_v, gsem).wait()
        pltpu.sync_copy(rows_v, agg_sh.at[dst_v.at[j]], add=True)
        return carry

    lax.fori_loop(0, CPW, body, 0)
    plsc.subcore_barrier()
    pltpu.sync_copy(agg_sh.at[pl.ds(s * RPT, RPT)],
                    out_hbm.at[c, pl.ds(s * RPT, RPT)])


def _mm_body(x_ref, w_ref, o_ref):
    o_ref[...] = jnp.dot(x_ref[...], w_ref[...],
                         preferred_element_type=jnp.float32)


def _scale_body(degp_ref, h_ref, dinv_ref, q1_ref):
    deg = degp_ref[0] + degp_ref[1] + 1.0
    dinv = lax.rsqrt(deg)
    dinv_ref[...] = dinv
    q1_ref[...] = h_ref[...] * dinv


def _mid_body(part_ref, q1_ref, dinv_ref, b1_ref, q2_ref):
    agg = part_ref[0] + part_ref[1] - q1_ref[...]
    dinv = dinv_ref[...]
    r = jnp.maximum(dinv * agg + b1_ref[...], 0.0)
    q2_ref[...] = dinv * r


def _final_body(part_ref, q2_ref, dinv_ref, w2_ref, b2_ref, o_ref):
    agg = part_ref[0] + part_ref[1] - q2_ref[...]
    g = dinv_ref[...] * agg
    z = jnp.dot(g, w2_ref[...], preferred_element_type=jnp.float32)
    z = z + b2_ref[...]
    m = jnp.max(z, axis=1, keepdims=True)
    lse = jnp.log(jnp.sum(jnp.exp(z - m), axis=1, keepdims=True)) + m
    o_ref[...] = z - lse


def _tc_call(body, out_shapes):
    return pl.pallas_call(body, out_shape=out_shapes)


def kernel(x, edge_index, W1, b1, W2, b2):
    src = edge_index[0].astype(jnp.int32)
    dst = edge_index[1].astype(jnp.int32)
    pad = jnp.full((E_PAD - E,), DUMMY, jnp.int32)
    src3 = jnp.concatenate([src, pad]).reshape(NW, CPW, CHUNK)
    dst3 = jnp.concatenate([dst, pad]).reshape(NW, CPW, CHUNK)
    x_p = jnp.pad(x, ((0, NPAD - N), (0, 0)))

    degp = _degree_kernel(dst3).reshape(NC, NPAD, 1)

    h = _tc_call(_mm_body, jax.ShapeDtypeStruct((NPAD, H), jnp.float32))(
        x_p, W1)

    dinv, q1 = _tc_call(
        _scale_body,
        (jax.ShapeDtypeStruct((NPAD, 1), jnp.float32),
         jax.ShapeDtypeStruct((NPAD, H), jnp.float32)))(degp, h)

    part1 = _edge_agg_kernel(q1, src3, dst3)

    q2 = _tc_call(_mid_body, jax.ShapeDtypeStruct((NPAD, H), jnp.float32))(
        part1, q1, dinv, b1.reshape(1, H))

    part2 = _edge_agg_kernel(q2, src3, dst3)

    out = _tc_call(_final_body, jax.ShapeDtypeStruct((NPAD, H), jnp.float32))(
        part2, q2, dinv, W2, b2.reshape(1, H))

    return out[:N]


# speedup vs baseline: 55.2971x; 1.0667x over previous
import functools

import jax
import jax.numpy as jnp
from jax import lax
from jax.experimental import pallas as pl
from jax.experimental.pallas import tpu as pltpu
from jax.experimental.pallas import tpu_sc as plsc

N = 10000
D = 128
H = 16
E = 320000

NC = 2
NS = 16
NW = NC * NS
CHUNK = 128
CPW = 80
EPW = CPW * CHUNK
E_PAD = NW * EPW
NPAD = 10240
RPT = NPAD // NS
DUMMY = 10008

_mesh = plsc.VectorSubcoreMesh(core_axis_name="c", subcore_axis_name="s")


@functools.partial(
    pl.kernel,
    out_type=jax.ShapeDtypeStruct((NC, NPAD), jnp.float32),
    mesh=_mesh,
    scratch_types=[
        pltpu.VMEM_SHARED((NPAD,), jnp.float32),
        pltpu.VMEM((CPW, CHUNK), jnp.int32),
        pltpu.VMEM((CHUNK,), jnp.float32),
        pltpu.VMEM((RPT,), jnp.float32),
    ],
)
def _degree_kernel(dst_hbm, out_hbm, deg_sh, dst_v, ones_v, zero_v):
    c = lax.axis_index("c")
    s = lax.axis_index("s")
    w = s * NC + c
    one = jnp.ones((16,), jnp.float32)
    zero = jnp.zeros((16,), jnp.float32)
    for k in range(CHUNK // 16):
        ones_v[pl.ds(k * 16, 16)] = one
    for k in range(RPT // 16):
        zero_v[pl.ds(k * 16, 16)] = zero
    pltpu.sync_copy(dst_hbm.at[w], dst_v)
    pltpu.sync_copy(zero_v, deg_sh.at[pl.ds(s * RPT, RPT)])
    plsc.subcore_barrier()

    def body(j, carry):
        pltpu.sync_copy(ones_v, deg_sh.at[dst_v.at[j]], add=True)
        return carry

    lax.fori_loop(0, CPW, body, 0)
    plsc.subcore_barrier()
    pltpu.sync_copy(deg_sh.at[pl.ds(s * RPT, RPT)],
                    out_hbm.at[c, pl.ds(s * RPT, RPT)])


@functools.partial(
    pl.kernel,
    out_type=jax.ShapeDtypeStruct((NC, NPAD, H), jnp.float32),
    mesh=_mesh,
    scratch_types=[
        pltpu.VMEM_SHARED((NPAD, H), jnp.float32),
        pltpu.VMEM_SHARED((NPAD, H), jnp.float32),
        pltpu.VMEM((CPW, CHUNK), jnp.int32),
        pltpu.VMEM((CPW, CHUNK), jnp.int32),
        pltpu.VMEM((CHUNK, H), jnp.float32),
        pltpu.VMEM((CHUNK, H), jnp.float32),
        pltpu.SemaphoreType.DMA,
        pltpu.SemaphoreType.DMA,
    ],
    compiler_params=pltpu.CompilerParams(use_tc_tiling_on_sc=False),
)
def _edge_agg_kernel(q_hbm, src_hbm, dst_hbm, out_hbm,
                     agg_sh, q_sh, src_v, dst_v, rows_a, rows_b,
                     sem_a, sem_b):
    c = lax.axis_index("c")
    s = lax.axis_index("s")
    w = s * NC + c
    pltpu.sync_copy(q_hbm.at[pl.ds(s * RPT, RPT)],
                    agg_sh.at[pl.ds(s * RPT, RPT)])
    pltpu.sync_copy(q_hbm.at[pl.ds(s * RPT, RPT)],
                    q_sh.at[pl.ds(s * RPT, RPT)])
    pltpu.sync_copy(src_hbm.at[w], src_v)
    pltpu.sync_copy(dst_hbm.at[w], dst_v)
    plsc.subcore_barrier()

    pltpu.async_copy(q_sh.at[src_v.at[0]], rows_a, sem_a)

    def body(i, carry):
        j = 2 * i
        pltpu.make_async_copy(q_sh.at[src_v.at[0]], rows_a, sem_a).wait()
        pltpu.async_copy(q_sh.at[src_v.at[j + 1]], rows_b, sem_b)
        pltpu.sync_copy(rows_a, agg_sh.at[dst_v.at[j]], add=True)
        pltpu.make_async_copy(q_sh.at[src_v.at[0]], rows_b, sem_b).wait()

        @pl.when(i < CPW // 2 - 1)
        def _():
            pltpu.async_copy(q_sh.at[src_v.at[j + 2]], rows_a, sem_a)

        pltpu.sync_copy(rows_b, agg_sh.at[dst_v.at[j + 1]], add=True)
        return carry

    lax.fori_loop(0, CPW // 2, body, 0)
    plsc.subcore_barrier()
    pltpu.sync_copy(agg_sh.at[pl.ds(s * RPT, RPT)],
                    out_hbm.at[c, pl.ds(s * RPT, RPT)])


def _mm_scale_body(x_ref, w_ref, degp_ref, dinv_ref, q1_ref):
    deg = degp_ref[0] + degp_ref[1] + 1.0
    dinv = lax.rsqrt(deg)
    dinv_ref[...] = dinv
    h = jnp.dot(x_ref[...], w_ref[...], preferred_element_type=jnp.float32)
    q1_ref[...] = h * dinv


def _mid_body(part_ref, q1_ref, dinv_ref, b1_ref, q2_ref):
    agg = part_ref[0] + part_ref[1] - q1_ref[...]
    dinv = dinv_ref[...]
    r = jnp.maximum(dinv * agg + b1_ref[...], 0.0)
    q2_ref[...] = dinv * r


def _final_body(part_ref, q2_ref, dinv_ref, w2_ref, b2_ref, o_ref):
    agg = part_ref[0] + part_ref[1] - q2_ref[...]
    g = dinv_ref[...] * agg
    z = jnp.dot(g, w2_ref[...], preferred_element_type=jnp.float32)
    z = z + b2_ref[...]
    m = jnp.max(z, axis=1, keepdims=True)
    lse = jnp.log(jnp.sum(jnp.exp(z - m), axis=1, keepdims=True)) + m
    o_ref[...] = z - lse


def _tc_call(body, out_shapes):
    return pl.pallas_call(body, out_shape=out_shapes)


def kernel(x, edge_index, W1, b1, W2, b2):
    src = edge_index[0].astype(jnp.int32)
    dst = edge_index[1].astype(jnp.int32)
    pad = jnp.full((E_PAD - E,), DUMMY, jnp.int32)
    src3 = jnp.concatenate([src, pad]).reshape(NW, CPW, CHUNK)
    dst3 = jnp.concatenate([dst, pad]).reshape(NW, CPW, CHUNK)
    x_p = jnp.pad(x, ((0, NPAD - N), (0, 0)))

    degp = _degree_kernel(dst3).reshape(NC, NPAD, 1)

    dinv, q1 = _tc_call(
        _mm_scale_body,
        (jax.ShapeDtypeStruct((NPAD, 1), jnp.float32),
         jax.ShapeDtypeStruct((NPAD, H), jnp.float32)))(x_p, W1, degp)

    part1 = _edge_agg_kernel(q1, src3, dst3)

    q2 = _tc_call(_mid_body, jax.ShapeDtypeStruct((NPAD, H), jnp.float32))(
        part1, q1, dinv, b1.reshape(1, H))

    part2 = _edge_agg_kernel(q2, src3, dst3)

    out = _tc_call(_final_body, jax.ShapeDtypeStruct((NPAD, H), jnp.float32))(
        part2, q2, dinv, W2, b2.reshape(1, H))

    return out[:N]


# speedup vs baseline: 59.6198x; 1.0782x over previous
import functools

import jax
import jax.numpy as jnp
from jax import lax
from jax.experimental import pallas as pl
from jax.experimental.pallas import tpu as pltpu
from jax.experimental.pallas import tpu_sc as plsc

N = 10000
D = 128
H = 16
E = 320000

NC = 2
NS = 16
NW = NC * NS
CHUNK = 128
CPW = 80
EPW = CPW * CHUNK
E_PAD = NW * EPW
NPAD = 10240
RPT = NPAD // NS
DUMMY = 10008
NBUF = 4

_mesh = plsc.VectorSubcoreMesh(core_axis_name="c", subcore_axis_name="s")
_params = pltpu.CompilerParams(use_tc_tiling_on_sc=False,
                               needs_layout_passes=False)


def _rsqrt_newton(d):
    i = lax.bitcast_convert_type(d, jnp.int32)
    i = 0x5F3759DF - lax.shift_right_arithmetic(i, 1)
    y = lax.bitcast_convert_type(i, jnp.float32)
    for _ in range(3):
        y = y * (1.5 - 0.5 * d * y * y)
    return y


def _edge_scatter(q_sh, agg_sh, src_v, dst_v, rows, gsems, ssems):
    for b in range(NBUF):
        pltpu.async_copy(q_sh.at[src_v.at[b]], rows[b], gsems[b])

    def body(i, carry):
        for b in range(NBUF):
            j = NBUF * i + b
            pltpu.make_async_copy(q_sh.at[src_v.at[0]], rows[b],
                                  gsems[b]).wait()
            pltpu.async_copy(rows[b], agg_sh.at[dst_v.at[j]], ssems[b],
                             add=True)
        for b in range(NBUF):
            @pl.when(i < CPW // NBUF - 1)
            def _():
                pltpu.make_async_copy(rows[b], agg_sh.at[dst_v.at[0]],
                                      ssems[b]).wait()
                pltpu.async_copy(q_sh.at[src_v.at[NBUF * i + NBUF + b]],
                                 rows[b], gsems[b])
        return carry

    lax.fori_loop(0, CPW // NBUF, body, 0)
    for b in range(NBUF):
        pltpu.make_async_copy(rows[b], agg_sh.at[dst_v.at[0]], ssems[b]).wait()


_SC_SCRATCH_COMMON = [
    pltpu.VMEM_SHARED((NPAD, H), jnp.float32),
    pltpu.VMEM_SHARED((NPAD, H), jnp.float32),
    pltpu.VMEM((CPW, CHUNK), jnp.int32),
    pltpu.VMEM((RPT, H), jnp.float32),
    pltpu.VMEM((RPT,), jnp.float32),
    *( [pltpu.VMEM((CHUNK, H), jnp.float32)] * NBUF ),
    *( [pltpu.SemaphoreType.DMA] * (2 * NBUF) ),
    pltpu.SemaphoreType.DMA,
]


@functools.partial(
    pl.kernel,
    out_type=(jax.ShapeDtypeStruct((NC, NPAD, H), jnp.float32),
              jax.ShapeDtypeStruct((NPAD, H), jnp.float32),
              jax.ShapeDtypeStruct((NPAD,), jnp.float32)),
    mesh=_mesh,
    scratch_types=[
        pltpu.VMEM_SHARED((NPAD,), jnp.float32),
        pltpu.VMEM((2, CPW, CHUNK), jnp.int32),
        pltpu.VMEM((CHUNK,), jnp.float32),
        pltpu.VMEM((RPT,), jnp.float32),
        *_SC_SCRATCH_COMMON,
    ],
    compiler_params=_params,
)
def _layer1_kernel(h_hbm, src_hbm, dst_hbm, part_out, q1_out, dinv_out,
                   deg_sh, dst2_v, ones_v, deg_v,
                   agg_sh, q_sh, src_v, slab_v, dinv_v, *bufs):
    rows = list(bufs[:NBUF])
    gsems = list(bufs[NBUF:2 * NBUF])
    ssems = list(bufs[2 * NBUF:3 * NBUF])
    hsem = bufs[3 * NBUF]
    c = lax.axis_index("c")
    s = lax.axis_index("s")
    w = s * NC + c
    one = jnp.ones((16,), jnp.float32)
    zero = jnp.zeros((16,), jnp.float32)
    for k in range(CHUNK // 16):
        ones_v[pl.ds(k * 16, 16)] = one
    for k in range(RPT // 16):
        deg_v[pl.ds(k * 16, 16)] = zero
    pltpu.async_copy(h_hbm.at[pl.ds(s * RPT, RPT)], slab_v, hsem)
    pltpu.sync_copy(src_hbm.at[w], src_v)
    pltpu.sync_copy(dst_hbm.at[pl.ds(2 * s, 2)], dst2_v)
    pltpu.sync_copy(deg_v, deg_sh.at[pl.ds(s * RPT, RPT)])
    plsc.subcore_barrier()

    def deg_body(i, carry):
        for k in range(NBUF):
            blk = k // 2
            col = 2 * i + (k % 2)
            @pl.when(i > 0)
            def _():
                pltpu.make_async_copy(ones_v, deg_sh.at[dst2_v.at[0, 0]],
                                      ssems[k]).wait()
            pltpu.async_copy(ones_v, deg_sh.at[dst2_v.at[blk, col]],
                             ssems[k], add=True)
        return carry

    lax.fori_loop(0, CPW // 2, deg_body, 0)
    for k in range(NBUF):
        pltpu.make_async_copy(ones_v, deg_sh.at[dst2_v.at[0, 0]],
                              ssems[k]).wait()
    plsc.subcore_barrier()

    pltpu.sync_copy(deg_sh.at[pl.ds(s * RPT, RPT)], deg_v)
    def dinv_body(i, carry):
        d = deg_v[pl.ds(i * 16, 16)] + 1.0
        dinv_v[pl.ds(i * 16, 16)] = _rsqrt_newton(d)
        return carry
    lax.fori_loop(0, RPT // 16, dinv_body, 0)
    pltpu.make_async_copy(h_hbm.at[pl.ds(s * RPT, RPT)], slab_v, hsem).wait()

    def scale_body(n, carry):
        dn = plsc.load_gather(dinv_v, [jnp.full((16,), n, jnp.int32)])
        slab_v[n, :] = slab_v[n, :] * dn
        return carry
    lax.fori_loop(0, RPT, scale_body, 0)

    pltpu.sync_copy(slab_v, q_sh.at[pl.ds(s * RPT, RPT)])
    pltpu.sync_copy(slab_v, agg_sh.at[pl.ds(s * RPT, RPT)])

    @pl.when(c == 0)
    def _():
        pltpu.sync_copy(slab_v, q1_out.at[pl.ds(s * RPT, RPT)])
        pltpu.sync_copy(dinv_v, dinv_out.at[pl.ds(s * RPT, RPT)])
    plsc.subcore_barrier()

    _edge_scatter(q_sh, agg_sh, src_v, dst2_v.at[c], rows, gsems, ssems)
    plsc.subcore_barrier()
    pltpu.sync_copy(agg_sh.at[pl.ds(s * RPT, RPT)],
                    part_out.at[c, pl.ds(s * RPT, RPT)])


@functools.partial(
    pl.kernel,
    out_type=(jax.ShapeDtypeStruct((NC, NPAD, H), jnp.float32),
              jax.ShapeDtypeStruct((NPAD, H), jnp.float32)),
    mesh=_mesh,
    scratch_types=[
        pltpu.VMEM((CPW, CHUNK), jnp.int32),
        pltpu.VMEM((RPT, H), jnp.float32),
        pltpu.VMEM((RPT, H), jnp.float32),
        pltpu.VMEM((16,), jnp.float32),
        pltpu.SemaphoreType.DMA,
        pltpu.SemaphoreType.DMA,
        pltpu.SemaphoreType.DMA,
        *_SC_SCRATCH_COMMON,
    ],
    compiler_params=_params,
)
def _layer2_kernel(part1_hbm, q1_hbm, dinv_hbm, b1_hbm, src_hbm, dst_hbm,
                   part_out, q2_out,
                   dst_v, p1_v, q1_v, b1_v, sem_a, sem_b, sem_c,
                   agg_sh, q_sh, src_v, slab_v, dinv_v, *bufs):
    rows = list(bufs[:NBUF])
    gsems = list(bufs[NBUF:2 * NBUF])
    ssems = list(bufs[2 * NBUF:3 * NBUF])
    dsem = bufs[3 * NBUF]
    c = lax.axis_index("c")
    s = lax.axis_index("s")
    w = s * NC + c
    pltpu.async_copy(part1_hbm.at[0, pl.ds(s * RPT, RPT)], slab_v, sem_a)
    pltpu.async_copy(part1_hbm.at[1, pl.ds(s * RPT, RPT)], p1_v, sem_b)
    pltpu.async_copy(q1_hbm.at[pl.ds(s * RPT, RPT)], q1_v, sem_c)
    pltpu.async_copy(dinv_hbm.at[pl.ds(s * RPT, RPT)], dinv_v, dsem)
    pltpu.sync_copy(b1_hbm, b1_v)
    pltpu.sync_copy(src_hbm.at[w], src_v)
    pltpu.sync_copy(dst_hbm.at[w], dst_v)
    pltpu.make_async_copy(part1_hbm.at[0, pl.ds(s * RPT, RPT)], slab_v,
                          sem_a).wait()
    pltpu.make_async_copy(part1_hbm.at[1, pl.ds(s * RPT, RPT)], p1_v,
                          sem_b).wait()
    pltpu.make_async_copy(q1_hbm.at[pl.ds(s * RPT, RPT)], q1_v, sem_c).wait()
    pltpu.make_async_copy(dinv_hbm.at[pl.ds(s * RPT, RPT)], dinv_v,
                          dsem).wait()
    b1r = b1_v[...]

    def mid_body(n, carry):
        dn = plsc.load_gather(dinv_v, [jnp.full((16,), n, jnp.int32)])
        agg = slab_v[n, :] + p1_v[n, :] - q1_v[n, :]
        r = jnp.maximum(dn * agg + b1r, 0.0)
        slab_v[n, :] = dn * r
        return carry
    lax.fori_loop(0, RPT, mid_body, 0)

    pltpu.sync_copy(slab_v, q_sh.at[pl.ds(s * RPT, RPT)])
    pltpu.sync_copy(slab_v, agg_sh.at[pl.ds(s * RPT, RPT)])

    @pl.when(c == 0)
    def _():
        pltpu.sync_copy(slab_v, q2_out.at[pl.ds(s * RPT, RPT)])
    plsc.subcore_barrier()

    _edge_scatter(q_sh, agg_sh, src_v, dst_v, rows, gsems, ssems)
    plsc.subcore_barrier()
    pltpu.sync_copy(agg_sh.at[pl.ds(s * RPT, RPT)],
                    part_out.at[c, pl.ds(s * RPT, RPT)])


def _mm_body(x_ref, w_ref, o_ref):
    o_ref[...] = jnp.dot(x_ref[...], w_ref[...],
                         preferred_element_type=jnp.float32)


def _final_body(part_ref, q2_ref, dinv_ref, w2_ref, b2_ref, o_ref):
    agg = part_ref[0] + part_ref[1] - q2_ref[...]
    g = dinv_ref[...] * agg
    z = jnp.dot(g, w2_ref[...], preferred_element_type=jnp.float32)
    z = z + b2_ref[...]
    m = jnp.max(z, axis=1, keepdims=True)
    lse = jnp.log(jnp.sum(jnp.exp(z - m), axis=1, keepdims=True)) + m
    o_ref[...] = z - lse


def kernel(x, edge_index, W1, b1, W2, b2):
    src = edge_index[0].astype(jnp.int32)
    dst = edge_index[1].astype(jnp.int32)
    pad = jnp.full((E_PAD - E,), DUMMY, jnp.int32)
    src3 = jnp.concatenate([src, pad]).reshape(NW, CPW, CHUNK)
    dst3 = jnp.concatenate([dst, pad]).reshape(NW, CPW, CHUNK)
    x_p = jnp.pad(x, ((0, NPAD - N), (0, 0)))

    h = pl.pallas_call(
        _mm_body, out_shape=jax.ShapeDtypeStruct((NPAD, H), jnp.float32))(
            x_p, W1)

    part1, q1, dinv = _layer1_kernel(h, src3, dst3)
    part2, q2 = _layer2_kernel(part1, q1, dinv, b1, src3, dst3)

    out = pl.pallas_call(
        _final_body, out_shape=jax.ShapeDtypeStruct((NPAD, H), jnp.float32))(
            part2, q2, dinv.reshape(NPAD, 1), W2, b2.reshape(1, H))

    return out[:N]


# speedup vs baseline: 60.2681x; 1.0109x over previous
import functools

import jax
import jax.numpy as jnp
from jax import lax
from jax.experimental import pallas as pl
from jax.experimental.pallas import tpu as pltpu
from jax.experimental.pallas import tpu_sc as plsc

N = 10000
D = 128
H = 16
E = 320000

NC = 2
NS = 16
NW = NC * NS
CHUNK = 128
CPW = 80
EPW = CPW * CHUNK
E_PAD = NW * EPW
NPAD = 10240
RPT = NPAD // NS
DUMMY = 10008
NBUF = 8

_mesh = plsc.VectorSubcoreMesh(core_axis_name="c", subcore_axis_name="s")
_params = pltpu.CompilerParams(use_tc_tiling_on_sc=False,
                               needs_layout_passes=False)


def _rsqrt_newton(d):
    i = lax.bitcast_convert_type(d, jnp.int32)
    i = 0x5F3759DF - lax.shift_right_arithmetic(i, 1)
    y = lax.bitcast_convert_type(i, jnp.float32)
    for _ in range(3):
        y = y * (1.5 - 0.5 * d * y * y)
    return y


def _edge_scatter(q_sh, agg_sh, src_v, dst_v, rows, gsems, ssems):
    for b in range(NBUF):
        pltpu.async_copy(q_sh.at[src_v.at[b]], rows[b], gsems[b])

    def body(i, carry):
        for b in range(NBUF):
            j = NBUF * i + b
            pltpu.make_async_copy(q_sh.at[src_v.at[0]], rows[b],
                                  gsems[b]).wait()
            pltpu.async_copy(rows[b], agg_sh.at[dst_v.at[j]], ssems[b],
                             add=True)
        for b in range(NBUF):
            @pl.when(i < CPW // NBUF - 1)
            def _():
                pltpu.make_async_copy(rows[b], agg_sh.at[dst_v.at[0]],
                                      ssems[b]).wait()
                pltpu.async_copy(q_sh.at[src_v.at[NBUF * i + NBUF + b]],
                                 rows[b], gsems[b])
        return carry

    lax.fori_loop(0, CPW // NBUF, body, 0)
    for b in range(NBUF):
        pltpu.make_async_copy(rows[b], agg_sh.at[dst_v.at[0]], ssems[b]).wait()


_SC_SCRATCH_COMMON = [
    pltpu.VMEM_SHARED((NPAD, H), jnp.float32),
    pltpu.VMEM_SHARED((NPAD, H), jnp.float32),
    pltpu.VMEM((CPW, CHUNK), jnp.int32),
    pltpu.VMEM((RPT, H), jnp.float32),
    pltpu.VMEM((RPT,), jnp.float32),
    *( [pltpu.VMEM((CHUNK, H), jnp.float32)] * NBUF ),
    *( [pltpu.SemaphoreType.DMA] * (2 * NBUF) ),
    pltpu.SemaphoreType.DMA,
]


@functools.partial(
    pl.kernel,
    out_type=(jax.ShapeDtypeStruct((NC, NPAD, H), jnp.float32),
              jax.ShapeDtypeStruct((NPAD, H), jnp.float32),
              jax.ShapeDtypeStruct((NPAD,), jnp.float32)),
    mesh=_mesh,
    scratch_types=[
        pltpu.VMEM_SHARED((NPAD,), jnp.float32),
        pltpu.VMEM((2, CPW, CHUNK), jnp.int32),
        pltpu.VMEM((CHUNK,), jnp.float32),
        pltpu.VMEM((RPT,), jnp.float32),
        *_SC_SCRATCH_COMMON,
    ],
    compiler_params=_params,
)
def _layer1_kernel(h_hbm, src_hbm, dst_hbm, part_out, q1_out, dinv_out,
                   deg_sh, dst2_v, ones_v, deg_v,
                   agg_sh, q_sh, src_v, slab_v, dinv_v, *bufs):
    rows = list(bufs[:NBUF])
    gsems = list(bufs[NBUF:2 * NBUF])
    ssems = list(bufs[2 * NBUF:3 * NBUF])
    hsem = bufs[3 * NBUF]
    c = lax.axis_index("c")
    s = lax.axis_index("s")
    w = s * NC + c
    one = jnp.ones((16,), jnp.float32)
    zero = jnp.zeros((16,), jnp.float32)
    for k in range(CHUNK // 16):
        ones_v[pl.ds(k * 16, 16)] = one
    for k in range(RPT // 16):
        deg_v[pl.ds(k * 16, 16)] = zero
    pltpu.async_copy(h_hbm.at[pl.ds(s * RPT, RPT)], slab_v, hsem)
    pltpu.sync_copy(src_hbm.at[w], src_v)
    pltpu.sync_copy(dst_hbm.at[pl.ds(2 * s, 2)], dst2_v)
    pltpu.sync_copy(deg_v, deg_sh.at[pl.ds(s * RPT, RPT)])
    plsc.subcore_barrier()

    def deg_body(i, carry):
        for k in range(4):
            blk = k // 2
            col = 2 * i + (k % 2)
            @pl.when(i > 0)
            def _():
                pltpu.make_async_copy(ones_v, deg_sh.at[dst2_v.at[0, 0]],
                                      ssems[k]).wait()
            pltpu.async_copy(ones_v, deg_sh.at[dst2_v.at[blk, col]],
                             ssems[k], add=True)
        return carry

    lax.fori_loop(0, CPW // 2, deg_body, 0)
    for k in range(4):
        pltpu.make_async_copy(ones_v, deg_sh.at[dst2_v.at[0, 0]],
                              ssems[k]).wait()
    plsc.subcore_barrier()

    pltpu.sync_copy(deg_sh.at[pl.ds(s * RPT, RPT)], deg_v)
    def dinv_body(i, carry):
        d = deg_v[pl.ds(i * 16, 16)] + 1.0
        dinv_v[pl.ds(i * 16, 16)] = _rsqrt_newton(d)
        return carry
    lax.fori_loop(0, RPT // 16, dinv_body, 0)
    pltpu.make_async_copy(h_hbm.at[pl.ds(s * RPT, RPT)], slab_v, hsem).wait()

    def scale_body(n, carry):
        dn = plsc.load_gather(dinv_v, [jnp.full((16,), n, jnp.int32)])
        slab_v[n, :] = slab_v[n, :] * dn
        return carry
    lax.fori_loop(0, RPT, scale_body, 0)

    pltpu.sync_copy(slab_v, q_sh.at[pl.ds(s * RPT, RPT)])
    pltpu.sync_copy(slab_v, agg_sh.at[pl.ds(s * RPT, RPT)])

    @pl.when(c == 0)
    def _():
        pltpu.sync_copy(slab_v, q1_out.at[pl.ds(s * RPT, RPT)])
        pltpu.sync_copy(dinv_v, dinv_out.at[pl.ds(s * RPT, RPT)])
    plsc.subcore_barrier()

    _edge_scatter(q_sh, agg_sh, src_v, dst2_v.at[c], rows, gsems, ssems)
    plsc.subcore_barrier()
    pltpu.sync_copy(agg_sh.at[pl.ds(s * RPT, RPT)],
                    part_out.at[c, pl.ds(s * RPT, RPT)])


@functools.partial(
    pl.kernel,
    out_type=(jax.ShapeDtypeStruct((NC, NPAD, H), jnp.float32),
              jax.ShapeDtypeStruct((NPAD, H), jnp.float32)),
    mesh=_mesh,
    scratch_types=[
        pltpu.VMEM((CPW, CHUNK), jnp.int32),
        pltpu.VMEM((RPT, H), jnp.float32),
        pltpu.VMEM((RPT, H), jnp.float32),
        pltpu.VMEM((16,), jnp.float32),
        pltpu.SemaphoreType.DMA,
        pltpu.SemaphoreType.DMA,
        pltpu.SemaphoreType.DMA,
        *_SC_SCRATCH_COMMON,
    ],
    compiler_params=_params,
)
def _layer2_kernel(part1_hbm, q1_hbm, dinv_hbm, b1_hbm, src_hbm, dst_hbm,
                   part_out, q2_out,
                   dst_v, p1_v, q1_v, b1_v, sem_a, sem_b, sem_c,
                   agg_sh, q_sh, src_v, slab_v, dinv_v, *bufs):
    rows = list(bufs[:NBUF])
    gsems = list(bufs[NBUF:2 * NBUF])
    ssems = list(bufs[2 * NBUF:3 * NBUF])
    dsem = bufs[3 * NBUF]
    c = lax.axis_index("c")
    s = lax.axis_index("s")
    w = s * NC + c
    pltpu.async_copy(part1_hbm.at[0, pl.ds(s * RPT, RPT)], slab_v, sem_a)
    pltpu.async_copy(part1_hbm.at[1, pl.ds(s * RPT, RPT)], p1_v, sem_b)
    pltpu.async_copy(q1_hbm.at[pl.ds(s * RPT, RPT)], q1_v, sem_c)
    pltpu.async_copy(dinv_hbm.at[pl.ds(s * RPT, RPT)], dinv_v, dsem)
    pltpu.sync_copy(b1_hbm, b1_v)
    pltpu.sync_copy(src_hbm.at[w], src_v)
    pltpu.sync_copy(dst_hbm.at[w], dst_v)
    pltpu.make_async_copy(part1_hbm.at[0, pl.ds(s * RPT, RPT)], slab_v,
                          sem_a).wait()
    pltpu.make_async_copy(part1_hbm.at[1, pl.ds(s * RPT, RPT)], p1_v,
                          sem_b).wait()
    pltpu.make_async_copy(q1_hbm.at[pl.ds(s * RPT, RPT)], q1_v, sem_c).wait()
    pltpu.make_async_copy(dinv_hbm.at[pl.ds(s * RPT, RPT)], dinv_v,
                          dsem).wait()
    b1r = b1_v[...]

    def mid_body(n, carry):
        dn = plsc.load_gather(dinv_v, [jnp.full((16,), n, jnp.int32)])
        agg = slab_v[n, :] + p1_v[n, :] - q1_v[n, :]
        r = jnp.maximum(dn * agg + b1r, 0.0)
        slab_v[n, :] = dn * r
        return carry
    lax.fori_loop(0, RPT, mid_body, 0)

    pltpu.sync_copy(slab_v, q_sh.at[pl.ds(s * RPT, RPT)])
    pltpu.sync_copy(slab_v, agg_sh.at[pl.ds(s * RPT, RPT)])

    @pl.when(c == 0)
    def _():
        pltpu.sync_copy(slab_v, q2_out.at[pl.ds(s * RPT, RPT)])
    plsc.subcore_barrier()

    _edge_scatter(q_sh, agg_sh, src_v, dst_v, rows, gsems, ssems)
    plsc.subcore_barrier()
    pltpu.sync_copy(agg_sh.at[pl.ds(s * RPT, RPT)],
                    part_out.at[c, pl.ds(s * RPT, RPT)])


def _mm_body(x_ref, w_ref, o_ref):
    o_ref[...] = jnp.dot(x_ref[...], w_ref[...],
                         preferred_element_type=jnp.float32)


def _final_body(part_ref, q2_ref, dinv_ref, w2_ref, b2_ref, o_ref):
    agg = part_ref[0] + part_ref[1] - q2_ref[...]
    g = dinv_ref[...] * agg
    z = jnp.dot(g, w2_ref[...], preferred_element_type=jnp.float32)
    z = z + b2_ref[...]
    m = jnp.max(z, axis=1, keepdims=True)
    lse = jnp.log(jnp.sum(jnp.exp(z - m), axis=1, keepdims=True)) + m
    o_ref[...] = z - lse


def kernel(x, edge_index, W1, b1, W2, b2):
    src = edge_index[0].astype(jnp.int32)
    dst = edge_index[1].astype(jnp.int32)
    pad = jnp.full((E_PAD - E,), DUMMY, jnp.int32)
    src3 = jnp.concatenate([src, pad]).reshape(NW, CPW, CHUNK)
    dst3 = jnp.concatenate([dst, pad]).reshape(NW, CPW, CHUNK)
    x_p = jnp.pad(x, ((0, NPAD - N), (0, 0)))

    h = pl.pallas_call(
        _mm_body, out_shape=jax.ShapeDtypeStruct((NPAD, H), jnp.float32))(
            x_p, W1)

    part1, q1, dinv = _layer1_kernel(h, src3, dst3)
    part2, q2 = _layer2_kernel(part1, q1, dinv, b1, src3, dst3)

    out = pl.pallas_call(
        _final_body, out_shape=jax.ShapeDtypeStruct((NPAD, H), jnp.float32))(
            part2, q2, dinv.reshape(NPAD, 1), W2, b2.reshape(1, H))

    return out[:N]
